# baseline (speedup 1.0000x reference)
import functools

import jax
import jax.numpy as jnp
from jax import lax
from jax.experimental import pallas as pl
from jax.experimental.pallas import tpu as pltpu
from jax.experimental.pallas import tpu_sc as plsc

N_NODES = 10000
N_EDGES = 160000
EB = 128
IN_FEATS = 256
H_FEATS = 512
NUM_CLASSES = 128

NC = 2
NS = 16
LANES = 16
C = 128
RING = 3


def _spmm_body(table, gixs, dst2, w2, out, acc, ixb, dtb, wtb,
               rows0, rows1, rows2, sg0, sg1, sg2, se0, se1, se2,
               *, fsplit, NR):
  c = lax.axis_index("c")
  s = lax.axis_index("s")
  big = 640
  last = N_NODES - big * (NS - 1)
  row0 = s * big
  br = s * NR if fsplit else (c * NS + s) * NR
  ses = (se0, se1, se2)
  rbs = (rows0, rows1, rows2)
  sgs = (sg0, sg1, sg2)

  def issue_edges(jchunk, q):
    pltpu.async_copy(gixs.at[c, br + jchunk], ixb.at[q], ses[q])
    pltpu.async_copy(dst2.at[br + jchunk], dtb.at[q], ses[q])
    pltpu.async_copy(w2.at[br + jchunk], wtb.at[q], ses[q])

  def wait_edges(q):
    pltpu.make_async_copy(gixs.at[c, br], ixb.at[q], ses[q]).wait()
    pltpu.make_async_copy(dst2.at[br], dtb.at[q], ses[q]).wait()
    pltpu.make_async_copy(w2.at[br], wtb.at[q], ses[q]).wait()

  def issue_gather(q, r):
    pltpu.async_copy(table.at[ixb.at[q]], rbs[r], sgs[r])

  def wait_gather(q, r):
    pltpu.make_async_copy(table.at[ixb.at[q]], rbs[r], sgs[r]).wait()

  for q in range(RING):
    issue_edges(q, q)

  def _zr_body(r, _):
    for k in range(C // LANES):
      rows0[r, pl.ds(k * LANES, LANES)] = jnp.zeros((LANES,), jnp.float32)
    return _
  lax.fori_loop(0, EB, _zr_body, None)

  nzblk = jnp.where(s == NS - 1, last // EB, big // EB)

  def _zcopy(j, _):
    pltpu.sync_copy(rows0, acc.at[pl.ds(row0 + j * EB, EB)])
    return _
  lax.fori_loop(0, nzblk, _zcopy, None)

  @pl.when(s == NS - 1)
  def _():
    pltpu.sync_copy(rows0.at[pl.ds(0, 16)],
                    acc.at[pl.ds(row0 + (last // EB) * EB, 16)])

  wait_edges(0)
  wait_edges(1)
  issue_gather(0, 0)
  issue_gather(1, 1)

  plsc.subcore_barrier()

  def _tri(j3, _):
    for u in range(RING):
      jj = j3 * RING + u
      q = u
      wait_gather(q, q)

      def _scale(g, _g):
        wv = wtb[q, pl.ds(g * LANES, LANES)]
        for l in range(LANES):
          wl = wv[l]
          for k in range(C // LANES):
            rbs[q][g * LANES + l, pl.ds(k * LANES, LANES)] = (
                rbs[q][g * LANES + l, pl.ds(k * LANES, LANES)] * wl)
        return _g
      lax.fori_loop(0, EB // LANES, _scale, None)

      pltpu.sync_copy(rbs[q], acc.at[dtb.at[q]], add=True)

      @pl.when(jj + RING < NR)
      def _():
        issue_edges(jj + RING, q)

      @pl.when(jj + 2 < NR)
      def _():
        wait_edges((q + 2) % RING)
        issue_gather((q + 2) % RING, (q + 2) % RING)
    return _
  lax.fori_loop(0, NR // RING, _tri, None)

  plsc.subcore_barrier()

  @pl.when(s < NS - 1)
  def _():
    pltpu.sync_copy(acc.at[pl.ds(row0, big)], out.at[c, pl.ds(row0, big)])

  @pl.when(s == NS - 1)
  def _():
    base = big * (NS - 1)
    pltpu.sync_copy(acc.at[pl.ds(base, last)], out.at[c, pl.ds(base, last)])


def _epad(fsplit):
  nwork = NS if fsplit else NC * NS
  unit = EB * nwork * RING
  return ((N_EDGES + unit - 1) // unit) * unit


def _make_spmm(fsplit):
  nwork = NS if fsplit else NC * NS
  NR = _epad(fsplit) // EB // nwork
  mesh = plsc.VectorSubcoreMesh(core_axis_name="c", subcore_axis_name="s")
  body = functools.partial(_spmm_body, fsplit=fsplit, NR=NR)
  return pl.kernel(
      body,
      out_type=jax.ShapeDtypeStruct((NC, N_NODES, C), jnp.float32),
      mesh=mesh,
      scratch_types=[
          pltpu.VMEM_SHARED((N_NODES, C), jnp.float32),
          pltpu.VMEM((RING, EB), jnp.int32),
          pltpu.VMEM((RING, EB), jnp.int32),
          pltpu.VMEM((RING, EB), jnp.float32),
          pltpu.VMEM((EB, C), jnp.float32),
          pltpu.VMEM((EB, C), jnp.float32),
          pltpu.VMEM((EB, C), jnp.float32),
          pltpu.SemaphoreType.DMA,
          pltpu.SemaphoreType.DMA,
          pltpu.SemaphoreType.DMA,
          pltpu.SemaphoreType.DMA,
          pltpu.SemaphoreType.DMA,
          pltpu.SemaphoreType.DMA,
      ],
  )


_spmm1 = _make_spmm(fsplit=True)
_spmm2 = _make_spmm(fsplit=False)


def _mlp_body(t1a_ref, t1b_ref, w1a_ref, w1b_ref, b1_ref, w2_ref, out_ref):
  h = jnp.dot(t1a_ref[...], w1a_ref[...], preferred_element_type=jnp.float32)
  h = h + jnp.dot(t1b_ref[...], w1b_ref[...],
                  preferred_element_type=jnp.float32)
  h = jnp.maximum(h + b1_ref[...], 0.0)
  out_ref[...] = jnp.dot(h, w2_ref[...], preferred_element_type=jnp.float32)


def _mlp(t1a, t1b, w1a, w1b, b1, w2t, R=400):
  n = t1a.shape[0]
  return pl.pallas_call(
      _mlp_body,
      grid=(n // R,),
      in_specs=[
          pl.BlockSpec((R, IN_FEATS // 2), lambda i: (i, 0)),
          pl.BlockSpec((R, IN_FEATS // 2), lambda i: (i, 0)),
          pl.BlockSpec((IN_FEATS // 2, H_FEATS), lambda i: (0, 0)),
          pl.BlockSpec((IN_FEATS // 2, H_FEATS), lambda i: (0, 0)),
          pl.BlockSpec((1, H_FEATS), lambda i: (0, 0)),
          pl.BlockSpec((H_FEATS, NUM_CLASSES), lambda i: (0, 0)),
      ],
      out_specs=pl.BlockSpec((R, NUM_CLASSES), lambda i: (i, 0)),
      out_shape=jax.ShapeDtypeStruct((n, NUM_CLASSES), jnp.float32),
  )(t1a, t1b, w1a, w1b, b1, w2t)


def _comb_body(p0_ref, p1_ref, b2_ref, out_ref):
  out_ref[...] = p0_ref[...] + p1_ref[...] + b2_ref[...]


def _combine(p0, p1, b2, R=1000):
  n = p0.shape[0]
  return pl.pallas_call(
      _comb_body,
      grid=(n // R,),
      in_specs=[
          pl.BlockSpec((R, NUM_CLASSES), lambda i: (i, 0)),
          pl.BlockSpec((R, NUM_CLASSES), lambda i: (i, 0)),
          pl.BlockSpec((1, NUM_CLASSES), lambda i: (0, 0)),
      ],
      out_specs=pl.BlockSpec((R, NUM_CLASSES), lambda i: (i, 0)),
      out_shape=jax.ShapeDtypeStruct((n, NUM_CLASSES), jnp.float32),
  )(p0, p1, b2)


def _pad_edges(arr, fsplit):
  return jnp.pad(arr, (0, _epad(fsplit) - N_EDGES)).reshape(-1, EB)


@jax.jit
def kernel(X, edge_index, edge_weight, W1, b1, W2, b2):
  src = edge_index[1]
  dst = edge_index[0]

  s1 = _pad_edges(src, True)
  gix1 = jnp.stack([2 * s1, 2 * s1 + 1])
  t1 = _spmm1(X.reshape(2 * N_NODES, IN_FEATS // 2), gix1,
              _pad_edges(dst, True), _pad_edges(edge_weight, True))

  w1t = W1.T
  g = _mlp(t1[0], t1[1], w1t[: IN_FEATS // 2], w1t[IN_FEATS // 2:],
           b1.reshape(1, H_FEATS), W2.T)

  s2 = _pad_edges(src, False)
  gix2 = jnp.stack([s2, s2])
  o2 = _spmm2(g, gix2, _pad_edges(dst, False), _pad_edges(edge_weight, False))

  return _combine(o2[0], o2[1], b2.reshape(1, NUM_CLASSES))


# speedup vs baseline: 1.3068x; 1.3068x over previous
import functools

import jax
import jax.numpy as jnp
from jax import lax
from jax.experimental import pallas as pl
from jax.experimental.pallas import tpu as pltpu
from jax.experimental.pallas import tpu_sc as plsc

N_NODES = 10000
N_EDGES = 160000
IN_FEATS = 256
H_FEATS = 512
NUM_CLASSES = 128

NC = 2
NS = 16
LANES = 16
C = 128

EB = 96
RING = 3
ER = 6
UNROLL = 6


def _spmm_body(table, gixs, dst2, w2, out, *refs, fsplit, NR):
  acc, ixb, dtb, wtb = refs[0], refs[1], refs[2], refs[3]
  rbs = refs[4:4 + RING]
  sgs = refs[4 + RING:4 + 2 * RING]
  ses = refs[4 + 2 * RING:4 + 2 * RING + ER]
  c = lax.axis_index("c")
  s = lax.axis_index("s")
  big = 640
  last = N_NODES - big * (NS - 1)
  row0 = s * big
  br = s * NR if fsplit else (c * NS + s) * NR

  def issue_edges(jchunk, q):
    pltpu.async_copy(gixs.at[c, br + jchunk], ixb.at[q], ses[q])
    pltpu.async_copy(dst2.at[br + jchunk], dtb.at[q], ses[q])
    pltpu.async_copy(w2.at[br + jchunk], wtb.at[q], ses[q])

  def wait_edges(q):
    pltpu.make_async_copy(gixs.at[c, br], ixb.at[q], ses[q]).wait()
    pltpu.make_async_copy(dst2.at[br], dtb.at[q], ses[q]).wait()
    pltpu.make_async_copy(w2.at[br], wtb.at[q], ses[q]).wait()

  def issue_gather(qe, qr):
    pltpu.async_copy(table.at[ixb.at[qe]], rbs[qr], sgs[qr])

  def wait_gather(qe, qr):
    pltpu.make_async_copy(table.at[ixb.at[qe]], rbs[qr], sgs[qr]).wait()

  for q in range(ER):
    issue_edges(q, q)

  ZB = 64

  def _zr_body(r, _):
    for k in range(C // LANES):
      rbs[0][r, pl.ds(k * LANES, LANES)] = jnp.zeros((LANES,), jnp.float32)
    return _
  lax.fori_loop(0, ZB, _zr_body, None)

  nzblk = jnp.where(s == NS - 1, last // ZB, big // ZB)
  zsrc = rbs[0].at[pl.ds(0, ZB)]

  def _zcopy(j, _):
    pltpu.sync_copy(zsrc, acc.at[pl.ds(row0 + j * ZB, ZB)])
    return _
  lax.fori_loop(0, nzblk, _zcopy, None)

  @pl.when(s == NS - 1)
  def _():
    pltpu.sync_copy(rbs[0].at[pl.ds(0, 16)],
                    acc.at[pl.ds(row0 + (last // ZB) * ZB, 16)])

  wait_edges(0)
  wait_edges(1)
  issue_gather(0, 0)
  issue_gather(1, 1 % RING)

  plsc.subcore_barrier()

  def _blk(jb, _):
    for u in range(UNROLL):
      jj = jb * UNROLL + u
      qe = u % ER
      qr = u % RING
      wait_gather(qe, qr)

      def _scale(g, _g):
        wv = wtb[qe, pl.ds(g * LANES, LANES)]
        for l in range(LANES):
          wl = wv[l]
          for k in range(C // LANES):
            rbs[qr][g * LANES + l, pl.ds(k * LANES, LANES)] = (
                rbs[qr][g * LANES + l, pl.ds(k * LANES, LANES)] * wl)
        return _g
      lax.fori_loop(0, EB // LANES, _scale, None)

      pltpu.sync_copy(rbs[qr], acc.at[dtb.at[qe]], add=True)

      @pl.when(jj + ER < NR)
      def _():
        issue_edges(jj + ER, qe)

      @pl.when(jj + 2 < NR)
      def _():
        wait_edges((u + 2) % ER)
        issue_gather((u + 2) % ER, (u + 2) % RING)
    return _
  lax.fori_loop(0, NR // UNROLL, _blk, None)

  plsc.subcore_barrier()

  @pl.when(s < NS - 1)
  def _():
    pltpu.sync_copy(acc.at[pl.ds(row0, big)], out.at[c, pl.ds(row0, big)])

  @pl.when(s == NS - 1)
  def _():
    base = big * (NS - 1)
    pltpu.sync_copy(acc.at[pl.ds(base, last)], out.at[c, pl.ds(base, last)])


def _epad():
  unit = EB * NC * NS * UNROLL
  return ((N_EDGES + unit - 1) // unit) * unit


def _make_spmm(fsplit):
  nwork = NS if fsplit else NC * NS
  NR = _epad() // EB // nwork
  assert NR % UNROLL == 0
  mesh = plsc.VectorSubcoreMesh(core_axis_name="c", subcore_axis_name="s")
  body = functools.partial(_spmm_body, fsplit=fsplit, NR=NR)
  scratch = [
      pltpu.VMEM_SHARED((N_NODES, C), jnp.float32),
      pltpu.VMEM((ER, EB), jnp.int32),
      pltpu.VMEM((ER, EB), jnp.int32),
      pltpu.VMEM((ER, EB), jnp.float32),
  ]
  scratch += [pltpu.VMEM((EB, C), jnp.float32) for _ in range(RING)]
  scratch += [pltpu.SemaphoreType.DMA for _ in range(RING + ER)]
  return pl.kernel(
      body,
      out_type=jax.ShapeDtypeStruct((NC, N_NODES, C), jnp.float32),
      mesh=mesh,
      scratch_types=scratch,
  )


_spmm1 = _make_spmm(fsplit=True)
_spmm2 = _make_spmm(fsplit=False)


def _mlp_body(t1a_ref, t1b_ref, w1a_ref, w1b_ref, b1_ref, w2_ref, out_ref):
  h = jnp.dot(t1a_ref[...], w1a_ref[...], preferred_element_type=jnp.float32)
  h = h + jnp.dot(t1b_ref[...], w1b_ref[...],
                  preferred_element_type=jnp.float32)
  h = jnp.maximum(h + b1_ref[...], 0.0)
  out_ref[...] = jnp.dot(h, w2_ref[...], preferred_element_type=jnp.float32)


def _mlp(t1a, t1b, w1a, w1b, b1, w2t, R=400):
  n = t1a.shape[0]
  return pl.pallas_call(
      _mlp_body,
      grid=(n // R,),
      in_specs=[
          pl.BlockSpec((R, IN_FEATS // 2), lambda i: (i, 0)),
          pl.BlockSpec((R, IN_FEATS // 2), lambda i: (i, 0)),
          pl.BlockSpec((IN_FEATS // 2, H_FEATS), lambda i: (0, 0)),
          pl.BlockSpec((IN_FEATS // 2, H_FEATS), lambda i: (0, 0)),
          pl.BlockSpec((1, H_FEATS), lambda i: (0, 0)),
          pl.BlockSpec((H_FEATS, NUM_CLASSES), lambda i: (0, 0)),
      ],
      out_specs=pl.BlockSpec((R, NUM_CLASSES), lambda i: (i, 0)),
      out_shape=jax.ShapeDtypeStruct((n, NUM_CLASSES), jnp.float32),
  )(t1a, t1b, w1a, w1b, b1, w2t)


def _comb_body(p0_ref, p1_ref, b2_ref, out_ref):
  out_ref[...] = p0_ref[...] + p1_ref[...] + b2_ref[...]


def _combine(p0, p1, b2, R=1000):
  n = p0.shape[0]
  return pl.pallas_call(
      _comb_body,
      grid=(n // R,),
      in_specs=[
          pl.BlockSpec((R, NUM_CLASSES), lambda i: (i, 0)),
          pl.BlockSpec((R, NUM_CLASSES), lambda i: (i, 0)),
          pl.BlockSpec((1, NUM_CLASSES), lambda i: (0, 0)),
      ],
      out_specs=pl.BlockSpec((R, NUM_CLASSES), lambda i: (i, 0)),
      out_shape=jax.ShapeDtypeStruct((n, NUM_CLASSES), jnp.float32),
  )(p0, p1, b2)


def _pad_edges(arr):
  return jnp.pad(arr, (0, _epad() - N_EDGES)).reshape(-1, EB)


@jax.jit
def kernel(X, edge_index, edge_weight, W1, b1, W2, b2):
  src = _pad_edges(edge_index[1])
  dst = _pad_edges(edge_index[0])
  ew = _pad_edges(edge_weight)

  gix1 = jnp.stack([2 * src, 2 * src + 1])
  t1 = _spmm1(X.reshape(2 * N_NODES, IN_FEATS // 2), gix1, dst, ew)

  w1t = W1.T
  g = _mlp(t1[0], t1[1], w1t[: IN_FEATS // 2], w1t[IN_FEATS // 2:],
           b1.reshape(1, H_FEATS), W2.T)

  gix2 = jnp.stack([src, src])
  o2 = _spmm2(g, gix2, dst, ew)

  return _combine(o2[0], o2[1], b2.reshape(1, NUM_CLASSES))


# speedup vs baseline: 1.7987x; 1.3764x over previous
import functools

import jax
import jax.numpy as jnp
from jax import lax
from jax.experimental import pallas as pl
from jax.experimental.pallas import tpu as pltpu
from jax.experimental.pallas import tpu_sc as plsc

N_NODES = 10000
N_EDGES = 160000
IN_FEATS = 256
H_FEATS = 512
NUM_CLASSES = 128

NC = 2
NS = 16
LANES = 16
C = 128

EB = 128
RING = 2
ER = 4
UNROLL = 4


def _spmm_body(table, gixs, dst2, w2, out, *refs, fsplit, NR):
  acc, ixb, dtb, wtb = refs[0], refs[1], refs[2], refs[3]
  rbs = refs[4:4 + RING]
  sgs = refs[4 + RING:4 + 2 * RING]
  ses = refs[4 + 2 * RING:4 + 2 * RING + ER]
  c = lax.axis_index("c")
  s = lax.axis_index("s")
  big = 640
  last = N_NODES - big * (NS - 1)
  row0 = s * big
  br = s * NR if fsplit else (c * NS + s) * NR

  def issue_edges(jchunk, q):
    pltpu.async_copy(gixs.at[c, br + jchunk], ixb.at[q], ses[q])
    pltpu.async_copy(dst2.at[br + jchunk], dtb.at[q], ses[q])
    pltpu.async_copy(w2.at[br + jchunk], wtb.at[q], ses[q])

  def wait_edges(q):
    pltpu.make_async_copy(gixs.at[c, br], ixb.at[q], ses[q]).wait()
    pltpu.make_async_copy(dst2.at[br], dtb.at[q], ses[q]).wait()
    pltpu.make_async_copy(w2.at[br], wtb.at[q], ses[q]).wait()

  def issue_gather(qe, qr):
    pltpu.async_copy(table.at[ixb.at[qe]], rbs[qr], sgs[qr])

  def wait_gather(qe, qr):
    pltpu.make_async_copy(table.at[ixb.at[qe]], rbs[qr], sgs[qr]).wait()

  for q in range(ER):
    issue_edges(q, q)

  ZB = 64

  def _zr_body(r, _):
    for k in range(C // LANES):
      rbs[0][r, pl.ds(k * LANES, LANES)] = jnp.zeros((LANES,), jnp.float32)
    return _
  lax.fori_loop(0, ZB, _zr_body, None)

  nzblk = jnp.where(s == NS - 1, last // ZB, big // ZB)
  zsrc = rbs[0].at[pl.ds(0, ZB)]

  def _zcopy(j, _):
    pltpu.sync_copy(zsrc, acc.at[pl.ds(row0 + j * ZB, ZB)])
    return _
  lax.fori_loop(0, nzblk, _zcopy, None)

  @pl.when(s == NS - 1)
  def _():
    pltpu.sync_copy(rbs[0].at[pl.ds(0, 16)],
                    acc.at[pl.ds(row0 + (last // ZB) * ZB, 16)])

  wait_edges(0)
  wait_edges(1)
  issue_gather(0, 0)
  issue_gather(1, 1 % RING)

  plsc.subcore_barrier()

  def _blk(jb, _):
    for u in range(UNROLL):
      jj = jb * UNROLL + u
      qe = u % ER
      qr = u % RING
      wait_gather(qe, qr)

      def _scale(g, _g):
        wv = wtb[qe, pl.ds(g * LANES, LANES)]
        for l in range(LANES):
          wl = wv[l]
          for k in range(C // LANES):
            rbs[qr][g * LANES + l, pl.ds(k * LANES, LANES)] = (
                rbs[qr][g * LANES + l, pl.ds(k * LANES, LANES)] * wl)
        return _g
      lax.fori_loop(0, EB // LANES, _scale, None)

      pltpu.sync_copy(rbs[qr], acc.at[dtb.at[qe]], add=True)

      @pl.when(jj + ER < NR)
      def _():
        issue_edges(jj + ER, qe)

      @pl.when(jj + 2 < NR)
      def _():
        wait_edges((u + 2) % ER)
        issue_gather((u + 2) % ER, (u + 2) % RING)
    return _
  lax.fori_loop(0, NR // UNROLL, _blk, None)

  plsc.subcore_barrier()

  @pl.when(s < NS - 1)
  def _():
    pltpu.sync_copy(acc.at[pl.ds(row0, big)], out.at[c, pl.ds(row0, big)])

  @pl.when(s == NS - 1)
  def _():
    base = big * (NS - 1)
    pltpu.sync_copy(acc.at[pl.ds(base, last)], out.at[c, pl.ds(base, last)])


def _epad():
  unit = EB * NC * NS * UNROLL
  return ((N_EDGES + unit - 1) // unit) * unit


def _make_spmm(fsplit):
  nwork = NS if fsplit else NC * NS
  NR = _epad() // EB // nwork
  assert NR % UNROLL == 0
  mesh = plsc.VectorSubcoreMesh(core_axis_name="c", subcore_axis_name="s")
  body = functools.partial(_spmm_body, fsplit=fsplit, NR=NR)
  scratch = [
      pltpu.VMEM_SHARED((N_NODES, C), jnp.float32),
      pltpu.VMEM((ER, EB), jnp.int32),
      pltpu.VMEM((ER, EB), jnp.int32),
      pltpu.VMEM((ER, EB), jnp.float32),
  ]
  scratch += [pltpu.VMEM((EB, C), jnp.float32) for _ in range(RING)]
  scratch += [pltpu.SemaphoreType.DMA for _ in range(RING + ER)]
  return pl.kernel(
      body,
      out_type=jax.ShapeDtypeStruct((NC, N_NODES, C), jnp.float32),
      mesh=mesh,
      scratch_types=scratch,
  )


_spmm1 = _make_spmm(fsplit=True)
_spmm2 = _make_spmm(fsplit=False)


def _mlp_body(t1a_ref, t1b_ref, w1a_ref, w1b_ref, b1_ref, w2_ref, out_ref):
  h = jnp.dot(t1a_ref[...], w1a_ref[...], preferred_element_type=jnp.float32)
  h = h + jnp.dot(t1b_ref[...], w1b_ref[...],
                  preferred_element_type=jnp.float32)
  h = jnp.maximum(h + b1_ref[...], 0.0)
  out_ref[...] = jnp.dot(h, w2_ref[...], preferred_element_type=jnp.float32)


def _mlp(t1a, t1b, w1a, w1b, b1, w2t, R=400):
  n = t1a.shape[0]
  return pl.pallas_call(
      _mlp_body,
      grid=(n // R,),
      in_specs=[
          pl.BlockSpec((R, IN_FEATS // 2), lambda i: (i, 0)),
          pl.BlockSpec((R, IN_FEATS // 2), lambda i: (i, 0)),
          pl.BlockSpec((IN_FEATS // 2, H_FEATS), lambda i: (0, 0)),
          pl.BlockSpec((IN_FEATS // 2, H_FEATS), lambda i: (0, 0)),
          pl.BlockSpec((1, H_FEATS), lambda i: (0, 0)),
          pl.BlockSpec((H_FEATS, NUM_CLASSES), lambda i: (0, 0)),
      ],
      out_specs=pl.BlockSpec((R, NUM_CLASSES), lambda i: (i, 0)),
      out_shape=jax.ShapeDtypeStruct((n, NUM_CLASSES), jnp.float32),
  )(t1a, t1b, w1a, w1b, b1, w2t)


def _comb_body(p0_ref, p1_ref, b2_ref, out_ref):
  out_ref[...] = p0_ref[...] + p1_ref[...] + b2_ref[...]


def _combine(p0, p1, b2, R=1000):
  n = p0.shape[0]
  return pl.pallas_call(
      _comb_body,
      grid=(n // R,),
      in_specs=[
          pl.BlockSpec((R, NUM_CLASSES), lambda i: (i, 0)),
          pl.BlockSpec((R, NUM_CLASSES), lambda i: (i, 0)),
          pl.BlockSpec((1, NUM_CLASSES), lambda i: (0, 0)),
      ],
      out_specs=pl.BlockSpec((R, NUM_CLASSES), lambda i: (i, 0)),
      out_shape=jax.ShapeDtypeStruct((n, NUM_CLASSES), jnp.float32),
  )(p0, p1, b2)


def _pad_edges(arr):
  return jnp.pad(arr, (0, _epad() - N_EDGES)).reshape(-1, EB)


@jax.jit
def kernel(X, edge_index, edge_weight, W1, b1, W2, b2):
  src = _pad_edges(edge_index[1])
  dst = _pad_edges(edge_index[0])
  ew = _pad_edges(edge_weight)

  gix1 = jnp.stack([2 * src, 2 * src + 1])
  t1 = _spmm1(X.reshape(2 * N_NODES, IN_FEATS // 2), gix1, dst, ew)

  w1t = W1.T
  g = _mlp(t1[0], t1[1], w1t[: IN_FEATS // 2], w1t[IN_FEATS // 2:],
           b1.reshape(1, H_FEATS), W2.T)

  gix2 = jnp.stack([src, src])
  o2 = _spmm2(g, gix2, dst, ew)

  return _combine(o2[0], o2[1], b2.reshape(1, NUM_CLASSES))


# speedup vs baseline: 1.9052x; 1.0592x over previous
import functools

import jax
import jax.numpy as jnp
from jax import lax
from jax.experimental import pallas as pl
from jax.experimental.pallas import tpu as pltpu
from jax.experimental.pallas import tpu_sc as plsc

N_NODES = 10000
N_EDGES = 160000
IN_FEATS = 256
H_FEATS = 512
NUM_CLASSES = 128

NC = 2
NS = 16
LANES = 16
C = 128

EB = 128
RING = 2
ER = 4
UNROLL = 4


def _spmm_body(table, gixs, dst2, w2, out, *refs, fsplit, NR):
  acc, ixb, dtb, wtb = refs[0], refs[1], refs[2], refs[3]
  rbs = refs[4:4 + RING]
  sgs = refs[4 + RING:4 + 2 * RING]
  ses = refs[4 + 2 * RING:4 + 2 * RING + ER]
  c = lax.axis_index("c")
  s = lax.axis_index("s")
  big = 640
  last = N_NODES - big * (NS - 1)
  row0 = s * big
  br = s * NR if fsplit else (c * NS + s) * NR

  def issue_edges(jchunk, q):
    pltpu.async_copy(gixs.at[c, br + jchunk], ixb.at[q], ses[q])
    pltpu.async_copy(dst2.at[br + jchunk], dtb.at[q], ses[q])
    pltpu.async_copy(w2.at[br + jchunk], wtb.at[q], ses[q])

  def wait_edges(q):
    pltpu.make_async_copy(gixs.at[c, br], ixb.at[q], ses[q]).wait()
    pltpu.make_async_copy(dst2.at[br], dtb.at[q], ses[q]).wait()
    pltpu.make_async_copy(w2.at[br], wtb.at[q], ses[q]).wait()

  def issue_gather(qe, qr):
    pltpu.async_copy(table.at[ixb.at[qe]], rbs[qr], sgs[qr])

  def wait_gather(qe, qr):
    pltpu.make_async_copy(table.at[ixb.at[qe]], rbs[qr], sgs[qr]).wait()

  for q in range(ER):
    issue_edges(q, q)

  ZB = 64

  def _zr_body(r, _):
    for k in range(C // LANES):
      rbs[0][r, pl.ds(k * LANES, LANES)] = jnp.zeros((LANES,), jnp.float32)
    return _
  lax.fori_loop(0, ZB, _zr_body, None)

  nzblk = jnp.where(s == NS - 1, last // ZB, big // ZB)
  zsrc = rbs[0].at[pl.ds(0, ZB)]

  def _zcopy(j, _):
    pltpu.sync_copy(zsrc, acc.at[pl.ds(row0 + j * ZB, ZB)])
    return _
  lax.fori_loop(0, nzblk, _zcopy, None)

  @pl.when(s == NS - 1)
  def _():
    pltpu.sync_copy(rbs[0].at[pl.ds(0, 16)],
                    acc.at[pl.ds(row0 + (last // ZB) * ZB, 16)])

  wait_edges(0)
  wait_edges(1)
  issue_gather(0, 0)
  issue_gather(1, 1 % RING)

  plsc.subcore_barrier()

  def _blk(jb, _):
    for u in range(UNROLL):
      jj = jb * UNROLL + u
      qe = u % ER
      qr = u % RING
      wait_gather(qe, qr)

      def _scale(g, _g):
        wv = wtb[qe, pl.ds(g * LANES, LANES)]
        for l in range(LANES):
          wl = wv[l]
          for k in range(C // LANES):
            rbs[qr][g * LANES + l, pl.ds(k * LANES, LANES)] = (
                rbs[qr][g * LANES + l, pl.ds(k * LANES, LANES)] * wl)
        return _g
      lax.fori_loop(0, EB // LANES, _scale, None)

      pltpu.sync_copy(rbs[qr], acc.at[dtb.at[qe]], add=True)

      @pl.when(jj + ER < NR)
      def _():
        issue_edges(jj + ER, qe)

      @pl.when(jj + 2 < NR)
      def _():
        wait_edges((u + 2) % ER)
        issue_gather((u + 2) % ER, (u + 2) % RING)
    return _
  lax.fori_loop(0, NR // UNROLL, _blk, None)

  plsc.subcore_barrier()

  @pl.when(s < NS - 1)
  def _():
    pltpu.sync_copy(acc.at[pl.ds(row0, big)], out.at[c, pl.ds(row0, big)])

  @pl.when(s == NS - 1)
  def _():
    base = big * (NS - 1)
    pltpu.sync_copy(acc.at[pl.ds(base, last)], out.at[c, pl.ds(base, last)])


def _epad():
  unit = EB * NC * NS * UNROLL
  return ((N_EDGES + unit - 1) // unit) * unit


def _make_spmm(fsplit):
  nwork = NS if fsplit else NC * NS
  NR = _epad() // EB // nwork
  assert NR % UNROLL == 0
  mesh = plsc.VectorSubcoreMesh(core_axis_name="c", subcore_axis_name="s")
  body = functools.partial(_spmm_body, fsplit=fsplit, NR=NR)
  scratch = [
      pltpu.VMEM_SHARED((N_NODES, C), jnp.float32),
      pltpu.VMEM((ER, EB), jnp.int32),
      pltpu.VMEM((ER, EB), jnp.int32),
      pltpu.VMEM((ER, EB), jnp.float32),
  ]
  scratch += [pltpu.VMEM((EB, C), jnp.float32) for _ in range(RING)]
  scratch += [pltpu.SemaphoreType.DMA for _ in range(RING + ER)]
  return pl.kernel(
      body,
      out_type=jax.ShapeDtypeStruct((NC, N_NODES, C), jnp.float32),
      mesh=mesh,
      scratch_types=scratch,
  )


_spmm1 = _make_spmm(fsplit=True)
_spmm2 = _make_spmm(fsplit=False)


def _mlp_body(t1a_ref, t1b_ref, w1a_ref, w1b_ref, b1_ref, w2_ref, out_ref):
  h = jnp.dot(t1a_ref[...], w1a_ref[...], preferred_element_type=jnp.float32)
  h = h + jnp.dot(t1b_ref[...], w1b_ref[...],
                  preferred_element_type=jnp.float32)
  h = jnp.maximum(h + b1_ref[...], 0.0)
  out_ref[...] = jnp.dot(h, w2_ref[...], preferred_element_type=jnp.float32)


def _mlp(t1a, t1b, w1a, w1b, b1, w2t, R=400):
  n = t1a.shape[0]
  return pl.pallas_call(
      _mlp_body,
      grid=(n // R,),
      in_specs=[
          pl.BlockSpec((R, IN_FEATS // 2), lambda i: (i, 0)),
          pl.BlockSpec((R, IN_FEATS // 2), lambda i: (i, 0)),
          pl.BlockSpec((IN_FEATS // 2, H_FEATS), lambda i: (0, 0)),
          pl.BlockSpec((IN_FEATS // 2, H_FEATS), lambda i: (0, 0)),
          pl.BlockSpec((1, H_FEATS), lambda i: (0, 0)),
          pl.BlockSpec((H_FEATS, NUM_CLASSES), lambda i: (0, 0)),
      ],
      out_specs=pl.BlockSpec((R, NUM_CLASSES), lambda i: (i, 0)),
      out_shape=jax.ShapeDtypeStruct((n, NUM_CLASSES), jnp.float32),
  )(t1a, t1b, w1a, w1b, b1, w2t)


def _comb_body(p0_ref, p1_ref, b2_ref, out_ref):
  out_ref[...] = p0_ref[...] + p1_ref[...] + b2_ref[...]


def _combine(p0, p1, b2, R=1000):
  n = p0.shape[0]
  return pl.pallas_call(
      _comb_body,
      grid=(n // R,),
      in_specs=[
          pl.BlockSpec((R, NUM_CLASSES), lambda i: (i, 0)),
          pl.BlockSpec((R, NUM_CLASSES), lambda i: (i, 0)),
          pl.BlockSpec((1, NUM_CLASSES), lambda i: (0, 0)),
      ],
      out_specs=pl.BlockSpec((R, NUM_CLASSES), lambda i: (i, 0)),
      out_shape=jax.ShapeDtypeStruct((n, NUM_CLASSES), jnp.float32),
  )(p0, p1, b2)


def _pad_edges(arr, spread=False):
  pad = _epad() - N_EDGES
  if spread:
    fill = jnp.arange(pad, dtype=arr.dtype) % N_NODES
    arr = jnp.concatenate([arr, fill])
  else:
    arr = jnp.pad(arr, (0, pad))
  return arr.reshape(-1, EB)


@jax.jit
def kernel(X, edge_index, edge_weight, W1, b1, W2, b2):
  src = _pad_edges(edge_index[1])
  dst = _pad_edges(edge_index[0], spread=True)
  ew = _pad_edges(edge_weight)

  gix1 = jnp.stack([2 * src, 2 * src + 1])
  t1 = _spmm1(X.reshape(2 * N_NODES, IN_FEATS // 2), gix1, dst, ew)

  w1t = W1.T
  g = _mlp(t1[0], t1[1], w1t[: IN_FEATS // 2], w1t[IN_FEATS // 2:],
           b1.reshape(1, H_FEATS), W2.T)

  gix2 = jnp.stack([src, src])
  o2 = _spmm2(g, gix2, dst, ew)

  return _combine(o2[0], o2[1], b2.reshape(1, NUM_CLASSES))


# speedup vs baseline: 1.9200x; 1.0078x over previous
import functools

import jax
import jax.numpy as jnp
from jax import lax
from jax.experimental import pallas as pl
from jax.experimental.pallas import tpu as pltpu
from jax.experimental.pallas import tpu_sc as plsc

N_NODES = 10000
N_EDGES = 160000
IN_FEATS = 256
H_FEATS = 512
NUM_CLASSES = 128

NC = 2
NS = 16
LANES = 16
C = 128

EB = 128
RING = 2
ER = 4
UNROLL = 4


def _spmm_body(table, gixs, dst2, w2, out, *refs, fsplit, NR):
  acc, ixb, dtb, wtb = refs[0], refs[1], refs[2], refs[3]
  rbs = refs[4:4 + RING]
  sgs = refs[4 + RING:4 + 2 * RING]
  ses = refs[4 + 2 * RING:4 + 2 * RING + ER]
  c = lax.axis_index("c")
  s = lax.axis_index("s")
  big = 640
  last = N_NODES - big * (NS - 1)
  row0 = s * big
  br = s * NR if fsplit else (c * NS + s) * NR

  def issue_edges(jchunk, q):
    pltpu.async_copy(gixs.at[c, br + jchunk], ixb.at[q], ses[q])
    pltpu.async_copy(dst2.at[br + jchunk], dtb.at[q], ses[q])
    pltpu.async_copy(w2.at[br + jchunk], wtb.at[q], ses[q])

  def wait_edges(q):
    pltpu.make_async_copy(gixs.at[c, br], ixb.at[q], ses[q]).wait()
    pltpu.make_async_copy(dst2.at[br], dtb.at[q], ses[q]).wait()
    pltpu.make_async_copy(w2.at[br], wtb.at[q], ses[q]).wait()

  def issue_gather(qe, qr):
    pltpu.async_copy(table.at[ixb.at[qe]], rbs[qr], sgs[qr])

  def wait_gather(qe, qr):
    pltpu.make_async_copy(table.at[ixb.at[qe]], rbs[qr], sgs[qr]).wait()

  for q in range(ER):
    issue_edges(q, q)

  ZB = 64

  def _zr_body(r, _):
    for k in range(C // LANES):
      rbs[0][r, pl.ds(k * LANES, LANES)] = jnp.zeros((LANES,), jnp.float32)
    return _
  lax.fori_loop(0, ZB, _zr_body, None)

  nzblk = jnp.where(s == NS - 1, last // ZB, big // ZB)
  zsrc = rbs[0].at[pl.ds(0, ZB)]

  def _zcopy(j, _):
    pltpu.sync_copy(zsrc, acc.at[pl.ds(row0 + j * ZB, ZB)])
    return _
  lax.fori_loop(0, nzblk, _zcopy, None)

  @pl.when(s == NS - 1)
  def _():
    pltpu.sync_copy(rbs[0].at[pl.ds(0, 16)],
                    acc.at[pl.ds(row0 + (last // ZB) * ZB, 16)])

  wait_edges(0)
  wait_edges(1)
  issue_gather(0, 0)
  issue_gather(1, 1 % RING)

  plsc.subcore_barrier()

  def _blk(jb, _):
    for u in range(UNROLL):
      jj = jb * UNROLL + u
      qe = u % ER
      qr = u % RING
      wait_gather(qe, qr)

      def _scale(g, _g):
        wv = wtb[qe, pl.ds(g * LANES, LANES)]
        for l in range(LANES):
          wl = wv[l]
          for k in range(C // LANES):
            rbs[qr][g * LANES + l, pl.ds(k * LANES, LANES)] = (
                rbs[qr][g * LANES + l, pl.ds(k * LANES, LANES)] * wl)
        return _g
      lax.fori_loop(0, EB // LANES, _scale, None)

      pltpu.sync_copy(rbs[qr], acc.at[dtb.at[qe]], add=True)

      @pl.when(jj + ER < NR)
      def _():
        issue_edges(jj + ER, qe)

      @pl.when(jj + 2 < NR)
      def _():
        wait_edges((u + 2) % ER)
        issue_gather((u + 2) % ER, (u + 2) % RING)
    return _
  lax.fori_loop(0, NR // UNROLL, _blk, None)

  plsc.subcore_barrier()

  @pl.when(s < NS - 1)
  def _():
    pltpu.sync_copy(acc.at[pl.ds(row0, big)], out.at[c, pl.ds(row0, big)])

  @pl.when(s == NS - 1)
  def _():
    base = big * (NS - 1)
    pltpu.sync_copy(acc.at[pl.ds(base, last)], out.at[c, pl.ds(base, last)])


def _epad():
  unit = EB * NC * NS * UNROLL
  return ((N_EDGES + unit - 1) // unit) * unit


def _make_spmm(fsplit):
  nwork = NS if fsplit else NC * NS
  NR = _epad() // EB // nwork
  assert NR % UNROLL == 0
  mesh = plsc.VectorSubcoreMesh(core_axis_name="c", subcore_axis_name="s")
  body = functools.partial(_spmm_body, fsplit=fsplit, NR=NR)
  scratch = [
      pltpu.VMEM_SHARED((N_NODES, C), jnp.float32),
      pltpu.VMEM((ER, EB), jnp.int32),
      pltpu.VMEM((ER, EB), jnp.int32),
      pltpu.VMEM((ER, EB), jnp.float32),
  ]
  scratch += [pltpu.VMEM((EB, C), jnp.float32) for _ in range(RING)]
  scratch += [pltpu.SemaphoreType.DMA for _ in range(RING + ER)]
  return pl.kernel(
      body,
      out_type=jax.ShapeDtypeStruct((NC, N_NODES, C), jnp.float32),
      mesh=mesh,
      scratch_types=scratch,
  )


_spmm1 = _make_spmm(fsplit=True)
_spmm2 = _make_spmm(fsplit=False)


def _mlp_body(t1a_ref, t1b_ref, w1a_ref, w1b_ref, b1_ref, w2_ref, out_ref):
  h = jnp.dot(t1a_ref[...], w1a_ref[...], preferred_element_type=jnp.float32)
  h = h + jnp.dot(t1b_ref[...], w1b_ref[...],
                  preferred_element_type=jnp.float32)
  h = jnp.maximum(h + b1_ref[...], 0.0)
  out_ref[...] = jnp.dot(h, w2_ref[...], preferred_element_type=jnp.float32)


def _mlp(t1a, t1b, w1a, w1b, b1, w2t, R=1000):
  n = t1a.shape[0]
  return pl.pallas_call(
      _mlp_body,
      grid=(n // R,),
      in_specs=[
          pl.BlockSpec((R, IN_FEATS // 2), lambda i: (i, 0)),
          pl.BlockSpec((R, IN_FEATS // 2), lambda i: (i, 0)),
          pl.BlockSpec((IN_FEATS // 2, H_FEATS), lambda i: (0, 0)),
          pl.BlockSpec((IN_FEATS // 2, H_FEATS), lambda i: (0, 0)),
          pl.BlockSpec((1, H_FEATS), lambda i: (0, 0)),
          pl.BlockSpec((H_FEATS, NUM_CLASSES), lambda i: (0, 0)),
      ],
      out_specs=pl.BlockSpec((R, NUM_CLASSES), lambda i: (i, 0)),
      out_shape=jax.ShapeDtypeStruct((n, NUM_CLASSES), jnp.float32),
  )(t1a, t1b, w1a, w1b, b1, w2t)


def _comb_body(p0_ref, p1_ref, b2_ref, out_ref):
  out_ref[...] = p0_ref[...] + p1_ref[...] + b2_ref[...]


def _combine(p0, p1, b2, R=1000):
  n = p0.shape[0]
  return pl.pallas_call(
      _comb_body,
      grid=(n // R,),
      in_specs=[
          pl.BlockSpec((R, NUM_CLASSES), lambda i: (i, 0)),
          pl.BlockSpec((R, NUM_CLASSES), lambda i: (i, 0)),
          pl.BlockSpec((1, NUM_CLASSES), lambda i: (0, 0)),
      ],
      out_specs=pl.BlockSpec((R, NUM_CLASSES), lambda i: (i, 0)),
      out_shape=jax.ShapeDtypeStruct((n, NUM_CLASSES), jnp.float32),
  )(p0, p1, b2)


def _pad_edges(arr, spread=False):
  pad = _epad() - N_EDGES
  if spread:
    fill = jnp.arange(pad, dtype=arr.dtype) % N_NODES
    arr = jnp.concatenate([arr, fill])
  else:
    arr = jnp.pad(arr, (0, pad))
  return arr.reshape(-1, EB)


@jax.jit
def kernel(X, edge_index, edge_weight, W1, b1, W2, b2):
  src = _pad_edges(edge_index[1])
  dst = _pad_edges(edge_index[0], spread=True)
  ew = _pad_edges(edge_weight)

  gix1 = jnp.stack([2 * src, 2 * src + 1])
  t1 = _spmm1(X.reshape(2 * N_NODES, IN_FEATS // 2), gix1, dst, ew)

  w1t = W1.T
  g = _mlp(t1[0], t1[1], w1t[: IN_FEATS // 2], w1t[IN_FEATS // 2:],
           b1.reshape(1, H_FEATS), W2.T)

  gix2 = jnp.stack([src, src])
  o2 = _spmm2(g, gix2, dst, ew)

  return _combine(o2[0], o2[1], b2.reshape(1, NUM_CLASSES))
